# baseline (speedup 1.0000x reference)
import functools

import jax
import jax.numpy as jnp
from jax import lax
from jax.experimental import pallas as pl
from jax.experimental.pallas import tpu as pltpu
from jax.experimental.pallas import tpu_sc as plsc

_EMBED = 64
_HID = 32


@functools.lru_cache(maxsize=None)
def _make_gather2(B: int, D: int):
    info = plsc.get_sparse_core_info()
    nw = info.num_cores * info.num_subcores
    assert B % (8 * nw) == 0
    bpw = B // nw
    mesh = plsc.VectorSubcoreMesh(core_axis_name="c", subcore_axis_name="s")

    @functools.partial(
        pl.kernel,
        mesh=mesh,
        compiler_params=pltpu.CompilerParams(needs_layout_passes=False, skip_device_barrier=True),
        out_type=(
            jax.ShapeDtypeStruct((B, D), jnp.float32),
            jax.ShapeDtypeStruct((B, D), jnp.float32),
        ),
        scratch_types=[
            pltpu.VMEM((bpw,), jnp.int32),
            pltpu.VMEM((bpw, D), jnp.float32),
            pltpu.SemaphoreType.DMA,
        ],
    )
    def gather2(ut, it, uid, iid, u_out, i_out, idx_v, rows, sem):
        wid = lax.axis_index("s") * info.num_cores + lax.axis_index("c")
        base = wid * bpw
        lanes = lax.iota(jnp.int32, 16)

        pltpu.sync_copy(uid.at[pl.ds(base, bpw)], idx_v)
        pltpu.sync_copy(ut.at[pl.ds(0, bpw)], rows)
        pltpu.sync_copy(rows, u_out.at[pl.ds(base, bpw)])
        for ids_hbm, tbl, out in ():
            pltpu.sync_copy(ids_hbm.at[pl.ds(base, bpw)], idx_v)

            def g_body(g, c, tbl=tbl):
                v = idx_v[pl.ds(g * 16, 16)]
                for l in range(16):
                    s = jnp.sum(jnp.where(lanes == l, v, 0))
                    pltpu.async_copy(tbl.at[s], rows.at[g * 16 + l], sem)
                return c

            lax.fori_loop(0, bpw // 16, g_body, 0)

            def d_body(j, c, tbl=tbl):
                pltpu.make_async_copy(tbl.at[0], rows.at[j], sem).wait()
                return c

            lax.fori_loop(0, bpw, d_body, 0)
            pltpu.sync_copy(rows, out.at[pl.ds(base, bpw)])

    return gather2


def _towers_body(u_ref, i_ref, uW1, ub1, uW2, ub2, iW1, ib1, iW2, ib2, out_ref):
    u = u_ref[...]
    uh = jnp.maximum(
        jnp.dot(u, uW1[...], preferred_element_type=jnp.float32) + ub1[...], 0.0
    )
    uv = jnp.dot(uh, uW2[...], preferred_element_type=jnp.float32) + ub2[...]
    it = i_ref[...]
    ih = jnp.maximum(
        jnp.dot(it, iW1[...], preferred_element_type=jnp.float32) + ib1[...], 0.0
    )
    iv = jnp.dot(ih, iW2[...], preferred_element_type=jnp.float32) + ib2[...]
    un = jnp.maximum(jnp.sqrt(jnp.sum(uv * uv, axis=1)), 1e-12)
    inn = jnp.maximum(jnp.sqrt(jnp.sum(iv * iv, axis=1)), 1e-12)
    out_ref[...] = jnp.sum(uv * iv, axis=1) / (un * inn)


@functools.lru_cache(maxsize=None)
def _make_towers(B: int, blk: int):
    grid = B // blk
    full = lambda shape: pl.BlockSpec(shape, lambda b: (0,) * len(shape))
    return pl.pallas_call(
        _towers_body,
        grid=(grid,),
        in_specs=[
            pl.BlockSpec((blk, _EMBED), lambda b: (b, 0)),
            pl.BlockSpec((blk, _EMBED), lambda b: (b, 0)),
            full((_EMBED, _HID)),
            full((1, _HID)),
            full((_HID, _HID)),
            full((1, _HID)),
            full((_EMBED, _HID)),
            full((1, _HID)),
            full((_HID, _HID)),
            full((1, _HID)),
        ],
        out_specs=pl.BlockSpec((blk,), lambda b: (b,)),
        out_shape=jax.ShapeDtypeStruct((B,), jnp.float32),
    )


def kernel(user_ids, item_ids, user_table, item_table,
                    uW1, ub1, uW2, ub2, iW1, ib1, iW2, ib2):
    B = user_ids.shape[0]
    D = user_table.shape[1]
    uid = user_ids.astype(jnp.int32)
    iid = item_ids.astype(jnp.int32)
    u_emb, i_emb = _make_gather2(B, D)(user_table, item_table, uid, iid)
    return jnp.sum(u_emb, axis=1) + jnp.sum(i_emb, axis=1)


def _kernel_tc_only(user_ids, item_ids, user_table, item_table,
                    uW1, ub1, uW2, ub2, iW1, ib1, iW2, ib2):
    B = user_ids.shape[0]
    u_emb = lax.dynamic_slice_in_dim(user_table, 0, B, 0)
    i_emb = lax.dynamic_slice_in_dim(item_table, 0, B, 0)
    towers = _make_towers(B, 2048)
    return towers(
        u_emb, i_emb,
        uW1, ub1.reshape(1, _HID), uW2, ub2.reshape(1, _HID),
        iW1, ib1.reshape(1, _HID), iW2, ib2.reshape(1, _HID),
    )


# speedup vs baseline: 1.1731x; 1.1731x over previous
import functools

import jax
import jax.numpy as jnp
from jax import lax
from jax.experimental import pallas as pl
from jax.experimental.pallas import tpu as pltpu
from jax.experimental.pallas import tpu_sc as plsc

_EMBED = 64
_HID = 32


def _project_body(tT_ref, W1, b1, W2, b2, out_ref):
    t = tT_ref[...]
    h = jnp.maximum(
        lax.dot_general(t, W1[...], (((0,), (0,)), ((), ())),
                        preferred_element_type=jnp.float32) + b1[...],
        0.0,
    )
    out_ref[...] = (
        jnp.dot(h, W2[...], preferred_element_type=jnp.float32) + b2[...]
    )


@functools.lru_cache(maxsize=None)
def _make_project(V: int, blk: int):
    grid = (V + blk - 1) // blk
    full = lambda shape: pl.BlockSpec(shape, lambda b: (0,) * len(shape))
    return pl.pallas_call(
        _project_body,
        grid=(grid,),
        in_specs=[
            pl.BlockSpec((_EMBED, blk), lambda b: (0, b)),
            full((_EMBED, _HID)),
            full((1, _HID)),
            full((_HID, _HID)),
            full((1, _HID)),
        ],
        out_specs=pl.BlockSpec((blk, _HID), lambda b: (b, 0)),
        out_shape=jax.ShapeDtypeStruct((V, _HID), jnp.float32),
    )


@functools.lru_cache(maxsize=None)
def _make_gather2(B: int, D: int):
    info = plsc.get_sparse_core_info()
    nw = info.num_cores * info.num_subcores
    assert B % (8 * nw) == 0
    bpw = B // nw
    mesh = plsc.VectorSubcoreMesh(core_axis_name="c", subcore_axis_name="s")

    @functools.partial(
        pl.kernel,
        mesh=mesh,
        compiler_params=pltpu.CompilerParams(needs_layout_passes=False),
        out_type=(
            jax.ShapeDtypeStruct((B, D), jnp.float32),
            jax.ShapeDtypeStruct((B, D), jnp.float32),
        ),
        scratch_types=[
            pltpu.VMEM((bpw,), jnp.int32),
            pltpu.VMEM((bpw, D), jnp.float32),
            pltpu.SemaphoreType.DMA,
        ],
    )
    def gather2(ut, it, uid, iid, u_out, i_out, idx_v, rows, sem):
        wid = lax.axis_index("s") * info.num_cores + lax.axis_index("c")
        base = wid * bpw
        lanes = lax.iota(jnp.int32, 16)

        for ids_hbm, tbl, out in ((uid, ut, u_out), (iid, it, i_out)):
            pltpu.sync_copy(ids_hbm.at[pl.ds(base, bpw)], idx_v)

            def g_body(g, c, tbl=tbl):
                v = idx_v[pl.ds(g * 16, 16)]
                for l in range(16):
                    s = jnp.sum(jnp.where(lanes == l, v, 0))
                    pltpu.async_copy(tbl.at[s], rows.at[g * 16 + l], sem)
                return c

            lax.fori_loop(0, bpw // 16, g_body, 0)

            def d_body(j, c, tbl=tbl):
                pltpu.make_async_copy(tbl.at[0], rows.at[j], sem).wait()
                return c

            lax.fori_loop(0, bpw, d_body, 0)
            pltpu.sync_copy(rows, out.at[pl.ds(base, bpw)])

    return gather2


def _normdot_body(u_ref, i_ref, out_ref):
    uv = u_ref[...]
    iv = i_ref[...]
    un = jnp.maximum(jnp.sqrt(jnp.sum(uv * uv, axis=1)), 1e-12)
    inn = jnp.maximum(jnp.sqrt(jnp.sum(iv * iv, axis=1)), 1e-12)
    out_ref[...] = jnp.sum(uv * iv, axis=1) / (un * inn)


@functools.lru_cache(maxsize=None)
def _make_normdot(B: int, blk: int):
    grid = B // blk
    return pl.pallas_call(
        _normdot_body,
        grid=(grid,),
        in_specs=[
            pl.BlockSpec((blk, _HID), lambda b: (b, 0)),
            pl.BlockSpec((blk, _HID), lambda b: (b, 0)),
        ],
        out_specs=pl.BlockSpec((blk,), lambda b: (b,)),
        out_shape=jax.ShapeDtypeStruct((B,), jnp.float32),
    )


def kernel(user_ids, item_ids, user_table, item_table,
           uW1, ub1, uW2, ub2, iW1, ib1, iW2, ib2):
    B = user_ids.shape[0]
    V = user_table.shape[0]
    uid = user_ids.astype(jnp.int32)
    iid = item_ids.astype(jnp.int32)
    project = _make_project(V, 8192)
    P_u = project(user_table.T, uW1, ub1.reshape(1, _HID), uW2, ub2.reshape(1, _HID))
    P_i = project(item_table.T, iW1, ib1.reshape(1, _HID), iW2, ib2.reshape(1, _HID))
    u_vec, i_vec = _make_gather2(B, _HID)(P_u, P_i, uid, iid)
    return _make_normdot(B, 2048)(u_vec, i_vec)


# speedup vs baseline: 1.3507x; 1.1514x over previous
import functools

import jax
import jax.numpy as jnp
from jax import lax
from jax.experimental import pallas as pl
from jax.experimental.pallas import tpu as pltpu
from jax.experimental.pallas import tpu_sc as plsc

_EMBED = 64
_HID = 32


def _project_body(tT_ref, W1, b1, out_ref):
    t = tT_ref[...].astype(jnp.bfloat16)
    w = W1[...].astype(jnp.bfloat16)
    out_ref[...] = (
        lax.dot_general(t, w, (((0,), (0,)), ((), ())),
                        preferred_element_type=jnp.float32) + b1[...]
    )


@functools.lru_cache(maxsize=None)
def _make_project(V: int, blk: int):
    grid = (V + blk - 1) // blk
    full = lambda shape: pl.BlockSpec(shape, lambda b: (0,) * len(shape))
    return pl.pallas_call(
        _project_body,
        grid=(grid,),
        in_specs=[
            pl.BlockSpec((_EMBED, blk), lambda b: (0, b)),
            full((_EMBED, _HID)),
            full((1, _HID)),
        ],
        out_specs=pl.BlockSpec((blk, _HID), lambda b: (b, 0)),
        out_shape=jax.ShapeDtypeStruct((V, _HID), jnp.float32),
    )


@functools.lru_cache(maxsize=None)
def _make_gather2(B: int, D: int):
    info = plsc.get_sparse_core_info()
    nw = info.num_cores * info.num_subcores
    assert B % (8 * nw) == 0
    bpw = B // nw
    mesh = plsc.VectorSubcoreMesh(core_axis_name="c", subcore_axis_name="s")

    @functools.partial(
        pl.kernel,
        mesh=mesh,
        compiler_params=pltpu.CompilerParams(needs_layout_passes=False),
        out_type=(
            jax.ShapeDtypeStruct((B, D), jnp.float32),
            jax.ShapeDtypeStruct((B, D), jnp.float32),
        ),
        scratch_types=[
            pltpu.VMEM((bpw,), jnp.int32),
            pltpu.VMEM((bpw, D), jnp.float32),
            pltpu.SemaphoreType.DMA,
        ],
    )
    def gather2(ut, it, uid, iid, u_out, i_out, idx_v, rows, sem):
        wid = lax.axis_index("s") * info.num_cores + lax.axis_index("c")
        base = wid * bpw
        lanes = lax.iota(jnp.int32, 16)

        for ids_hbm, tbl, out in ((uid, ut, u_out), (iid, it, i_out)):
            pltpu.sync_copy(ids_hbm.at[pl.ds(base, bpw)], idx_v)

            def g_body(g, c, tbl=tbl):
                v = idx_v[pl.ds(g * 16, 16)]
                for l in range(16):
                    s = jnp.sum(jnp.where(lanes == l, v, 0))
                    pltpu.async_copy(tbl.at[s], rows.at[g * 16 + l], sem)
                return c

            lax.fori_loop(0, bpw // 16, g_body, 0)

            def d_body(j, c, tbl=tbl):
                pltpu.make_async_copy(tbl.at[0], rows.at[j], sem).wait()
                return c

            lax.fori_loop(0, bpw, d_body, 0)
            pltpu.sync_copy(rows, out.at[pl.ds(base, bpw)])

    return gather2


def _finish_body(u_ref, i_ref, uW2, ub2, iW2, ib2, out_ref):
    uh = jnp.maximum(u_ref[...], 0.0)
    uv = jnp.dot(uh, uW2[...], preferred_element_type=jnp.float32) + ub2[...]
    ih = jnp.maximum(i_ref[...], 0.0)
    iv = jnp.dot(ih, iW2[...], preferred_element_type=jnp.float32) + ib2[...]
    un = jnp.maximum(jnp.sqrt(jnp.sum(uv * uv, axis=1)), 1e-12)
    inn = jnp.maximum(jnp.sqrt(jnp.sum(iv * iv, axis=1)), 1e-12)
    out_ref[...] = jnp.sum(uv * iv, axis=1) / (un * inn)


@functools.lru_cache(maxsize=None)
def _make_finish(B: int, blk: int):
    grid = B // blk
    full = lambda shape: pl.BlockSpec(shape, lambda b: (0,) * len(shape))
    return pl.pallas_call(
        _finish_body,
        grid=(grid,),
        in_specs=[
            pl.BlockSpec((blk, _HID), lambda b: (b, 0)),
            pl.BlockSpec((blk, _HID), lambda b: (b, 0)),
            full((_HID, _HID)),
            full((1, _HID)),
            full((_HID, _HID)),
            full((1, _HID)),
        ],
        out_specs=pl.BlockSpec((blk,), lambda b: (b,)),
        out_shape=jax.ShapeDtypeStruct((B,), jnp.float32),
    )


def kernel(user_ids, item_ids, user_table, item_table,
           uW1, ub1, uW2, ub2, iW1, ib1, iW2, ib2):
    B = user_ids.shape[0]
    V = user_table.shape[0]
    uid = user_ids.astype(jnp.int32)
    iid = item_ids.astype(jnp.int32)
    project = _make_project(V, 32768)
    P_u = project(user_table.T, uW1, ub1.reshape(1, _HID))
    P_i = project(item_table.T, iW1, ib1.reshape(1, _HID))
    u1, i1 = _make_gather2(B, _HID)(P_u, P_i, uid, iid)
    return _make_finish(B, 2048)(
        u1, i1,
        uW2, ub2.reshape(1, _HID), iW2, ib2.reshape(1, _HID),
    )


# speedup vs baseline: 1.6056x; 1.1887x over previous
import functools

import jax
import jax.numpy as jnp
from jax import lax
from jax.experimental import pallas as pl
from jax.experimental.pallas import tpu as pltpu
from jax.experimental.pallas import tpu_sc as plsc

_EMBED = 64
_HID = 32
_BLK = 32768
_BLKQ = _BLK // 4


def _project_body(tT_ref, W1, b1, out_ref):
    w = W1[...].astype(jnp.bfloat16)
    parts = []
    for c in range(4):
        t = tT_ref[:, c * _BLKQ:(c + 1) * _BLKQ].astype(jnp.bfloat16)
        parts.append(
            lax.dot_general(t, w, (((0,), (0,)), ((), ())),
                            preferred_element_type=jnp.float32) + b1[...]
        )
    out_ref[...] = jnp.concatenate(parts, axis=1)


@functools.lru_cache(maxsize=None)
def _make_project(V: int):
    grid = (V + _BLK - 1) // _BLK
    full = lambda shape: pl.BlockSpec(shape, lambda b: (0,) * len(shape))
    return pl.pallas_call(
        _project_body,
        grid=(grid,),
        in_specs=[
            pl.BlockSpec((_EMBED, _BLK), lambda b: (0, b)),
            full((_EMBED, _HID)),
            full((1, _HID)),
        ],
        out_specs=pl.BlockSpec((_BLKQ, 4 * _HID), lambda b: (b, 0)),
        out_shape=jax.ShapeDtypeStruct((grid * _BLKQ, 4 * _HID), jnp.float32),
    )


@functools.lru_cache(maxsize=None)
def _make_gather2(B: int, D: int, V: int):
    info = plsc.get_sparse_core_info()
    nw = info.num_cores * info.num_subcores
    assert B % (8 * nw) == 0
    bpw = B // nw
    mesh = plsc.VectorSubcoreMesh(core_axis_name="c", subcore_axis_name="s")

    @functools.partial(
        pl.kernel,
        mesh=mesh,
        compiler_params=pltpu.CompilerParams(needs_layout_passes=False),
        out_type=(
            jax.ShapeDtypeStruct((B, D), jnp.float32),
            jax.ShapeDtypeStruct((B, D), jnp.float32),
        ),
        scratch_types=[
            pltpu.VMEM((bpw,), jnp.int32),
            pltpu.VMEM((bpw, D), jnp.float32),
            pltpu.SemaphoreType.DMA,
        ],
    )
    def gather2(ut, it, uid, iid, u_out, i_out, idx_v, rows, sem):
        wid = lax.axis_index("s") * info.num_cores + lax.axis_index("c")
        base = wid * bpw
        lanes = lax.iota(jnp.int32, 16)

        for ids_hbm, tbl, out in ((uid, ut, u_out), (iid, it, i_out)):
            pltpu.sync_copy(ids_hbm.at[pl.ds(base, bpw)], idx_v)

            def g_body(g, c, tbl=tbl):
                v = idx_v[pl.ds(g * 16, 16)]
                for l in range(16):
                    s = jnp.sum(jnp.where(lanes == l, v, 0))
                    pltpu.async_copy(tbl.at[s], rows.at[g * 16 + l], sem)
                return c

            lax.fori_loop(0, bpw // 16, g_body, 0)

            def d_body(j, c, tbl=tbl):
                pltpu.make_async_copy(tbl.at[0], rows.at[j], sem).wait()
                return c

            lax.fori_loop(0, bpw, d_body, 0)
            pltpu.sync_copy(rows, out.at[pl.ds(base, bpw)])

    return gather2


def _finish_body(u4_ref, i4_ref, us_ref, is_ref, uW2, ub2, iW2, ib2, out_ref):
    def select(p4, slot):
        acc = jnp.zeros_like(p4[:, :_HID])
        for c in range(4):
            acc = acc + jnp.where(slot == c, p4[:, c * _HID:(c + 1) * _HID], 0.0)
        return acc

    uh = jnp.maximum(select(u4_ref[...], us_ref[...]), 0.0)
    uv = jnp.dot(uh, uW2[...], preferred_element_type=jnp.float32) + ub2[...]
    ih = jnp.maximum(select(i4_ref[...], is_ref[...]), 0.0)
    iv = jnp.dot(ih, iW2[...], preferred_element_type=jnp.float32) + ib2[...]
    un = jnp.maximum(jnp.sqrt(jnp.sum(uv * uv, axis=1)), 1e-12)
    inn = jnp.maximum(jnp.sqrt(jnp.sum(iv * iv, axis=1)), 1e-12)
    out_ref[...] = jnp.sum(uv * iv, axis=1) / (un * inn)


@functools.lru_cache(maxsize=None)
def _make_finish(B: int, blk: int):
    grid = B // blk
    full = lambda shape: pl.BlockSpec(shape, lambda b: (0,) * len(shape))
    return pl.pallas_call(
        _finish_body,
        grid=(grid,),
        in_specs=[
            pl.BlockSpec((blk, 4 * _HID), lambda b: (b, 0)),
            pl.BlockSpec((blk, 4 * _HID), lambda b: (b, 0)),
            pl.BlockSpec((blk, 1), lambda b: (b, 0)),
            pl.BlockSpec((blk, 1), lambda b: (b, 0)),
            full((_HID, _HID)),
            full((1, _HID)),
            full((_HID, _HID)),
            full((1, _HID)),
        ],
        out_specs=pl.BlockSpec((blk,), lambda b: (b,)),
        out_shape=jax.ShapeDtypeStruct((B,), jnp.float32),
    )


def kernel(user_ids, item_ids, user_table, item_table,
           uW1, ub1, uW2, ub2, iW1, ib1, iW2, ib2):
    B = user_ids.shape[0]
    V = user_table.shape[0]
    uid = user_ids.astype(jnp.int32)
    iid = item_ids.astype(jnp.int32)
    urow = (uid // _BLK) * _BLKQ + (uid % _BLKQ)
    irow = (iid // _BLK) * _BLKQ + (iid % _BLKQ)
    uslot = (uid % _BLK) // _BLKQ
    islot = (iid % _BLK) // _BLKQ
    project = _make_project(V)
    P_u = project(user_table.T, uW1, ub1.reshape(1, _HID))
    P_i = project(item_table.T, iW1, ib1.reshape(1, _HID))
    u4, i4 = _make_gather2(B, 4 * _HID, V)(P_u, P_i, urow, irow)
    return _make_finish(B, 2048)(
        u4, i4, uslot.reshape(B, 1), islot.reshape(B, 1),
        uW2, ub2.reshape(1, _HID), iW2, ib2.reshape(1, _HID),
    )


# speedup vs baseline: 1.6193x; 1.0085x over previous
import functools

import jax
import jax.numpy as jnp
from jax import lax
from jax.experimental import pallas as pl
from jax.experimental.pallas import tpu as pltpu
from jax.experimental.pallas import tpu_sc as plsc

_EMBED = 64
_HID = 32
_BLK = 32768
_BLKQ = _BLK // 4


def _project_body(tT_ref, W1, b1, out_ref):
    w = W1[...].astype(jnp.bfloat16)
    parts = []
    for c in range(4):
        t = tT_ref[:, c * _BLKQ:(c + 1) * _BLKQ].astype(jnp.bfloat16)
        parts.append(
            lax.dot_general(t, w, (((0,), (0,)), ((), ())),
                            preferred_element_type=jnp.float32) + b1[...]
        )
    out_ref[...] = jnp.concatenate(parts, axis=1)


@functools.lru_cache(maxsize=None)
def _make_project(V: int):
    grid = (V + _BLK - 1) // _BLK
    full = lambda shape: pl.BlockSpec(shape, lambda b: (0,) * len(shape))
    return pl.pallas_call(
        _project_body,
        grid=(grid,),
        in_specs=[
            pl.BlockSpec((_EMBED, _BLK), lambda b: (0, b)),
            full((_EMBED, _HID)),
            full((1, _HID)),
        ],
        out_specs=pl.BlockSpec((_BLKQ, 4 * _HID), lambda b: (b, 0)),
        out_shape=jax.ShapeDtypeStruct((grid * _BLKQ, 4 * _HID), jnp.float32),
    )


@functools.lru_cache(maxsize=None)
def _make_gather(B: int, D: int, V: int):
    info = plsc.get_sparse_core_info()
    nw = info.num_cores * info.num_subcores
    assert B % (8 * nw) == 0
    bpw = B // nw
    mesh = plsc.VectorSubcoreMesh(core_axis_name="c", subcore_axis_name="s")

    @functools.partial(
        pl.kernel,
        mesh=mesh,
        compiler_params=pltpu.CompilerParams(needs_layout_passes=False),
        out_type=jax.ShapeDtypeStruct((B, D), jnp.float32),
        scratch_types=[
            pltpu.VMEM((bpw,), jnp.int32),
            pltpu.VMEM((bpw, D), jnp.float32),
            pltpu.SemaphoreType.DMA,
        ],
    )
    def gather1(tbl, ids_hbm, out, idx_v, rows, sem):
        wid = lax.axis_index("s") * info.num_cores + lax.axis_index("c")
        base = wid * bpw
        lanes = lax.iota(jnp.int32, 16)
        pltpu.sync_copy(ids_hbm.at[pl.ds(base, bpw)], idx_v)

        def g_body(g, c):
            v = idx_v[pl.ds(g * 16, 16)]
            for l in range(16):
                s = jnp.sum(jnp.where(lanes == l, v, 0))
                pltpu.async_copy(tbl.at[s], rows.at[g * 16 + l], sem)
            return c

        lax.fori_loop(0, bpw // 16, g_body, 0)

        def d_body(j, c):
            pltpu.make_async_copy(tbl.at[0], rows.at[j], sem).wait()
            return c

        lax.fori_loop(0, bpw, d_body, 0)
        pltpu.sync_copy(rows, out.at[pl.ds(base, bpw)])

    return gather1


def _finish_body(u4_ref, i4_ref, us_ref, is_ref, uW2, ub2, iW2, ib2, out_ref):
    def select(p4, slot):
        acc = jnp.zeros(p4[:, :_HID].shape, jnp.float32)
        for c in range(4):
            sel = p4[:, c * _HID:(c + 1) * _HID].astype(jnp.float32)
            acc = acc + jnp.where(slot == c, sel, 0.0)
        return acc

    uh = jnp.maximum(select(u4_ref[...], us_ref[...]), 0.0)
    uv = jnp.dot(uh, uW2[...], preferred_element_type=jnp.float32) + ub2[...]
    ih = jnp.maximum(select(i4_ref[...], is_ref[...]), 0.0)
    iv = jnp.dot(ih, iW2[...], preferred_element_type=jnp.float32) + ib2[...]
    un = jnp.maximum(jnp.sqrt(jnp.sum(uv * uv, axis=1)), 1e-12)
    inn = jnp.maximum(jnp.sqrt(jnp.sum(iv * iv, axis=1)), 1e-12)
    out_ref[...] = jnp.sum(uv * iv, axis=1) / (un * inn)


@functools.lru_cache(maxsize=None)
def _make_finish(B: int, blk: int):
    grid = B // blk
    full = lambda shape: pl.BlockSpec(shape, lambda b: (0,) * len(shape))
    return pl.pallas_call(
        _finish_body,
        grid=(grid,),
        in_specs=[
            pl.BlockSpec((blk, 4 * _HID), lambda b: (b, 0)),
            pl.BlockSpec((blk, 4 * _HID), lambda b: (b, 0)),
            pl.BlockSpec((blk, 1), lambda b: (b, 0)),
            pl.BlockSpec((blk, 1), lambda b: (b, 0)),
            full((_HID, _HID)),
            full((1, _HID)),
            full((_HID, _HID)),
            full((1, _HID)),
        ],
        out_specs=pl.BlockSpec((blk,), lambda b: (b,)),
        out_shape=jax.ShapeDtypeStruct((B,), jnp.float32),
    )


def kernel(user_ids, item_ids, user_table, item_table,
           uW1, ub1, uW2, ub2, iW1, ib1, iW2, ib2):
    B = user_ids.shape[0]
    V = user_table.shape[0]
    uid = user_ids.astype(jnp.int32)
    iid = item_ids.astype(jnp.int32)
    urow = (uid // _BLK) * _BLKQ + (uid % _BLKQ)
    irow = (iid // _BLK) * _BLKQ + (iid % _BLKQ)
    uslot = (uid % _BLK) // _BLKQ
    islot = (iid % _BLK) // _BLKQ
    project = _make_project(V)
    gather = _make_gather(B, 4 * _HID, V)
    P_u = project(user_table.T, uW1, ub1.reshape(1, _HID))
    u4 = gather(P_u, urow)
    P_i = project(item_table.T, iW1, ib1.reshape(1, _HID))
    i4 = gather(P_i, irow)
    return _make_finish(B, 2048)(
        u4, i4, uslot.reshape(B, 1), islot.reshape(B, 1),
        uW2, ub2.reshape(1, _HID), iW2, ib2.reshape(1, _HID),
    )


# speedup vs baseline: 1.8962x; 1.1710x over previous
import functools

import jax
import jax.numpy as jnp
from jax import lax
from jax.experimental import pallas as pl
from jax.experimental.pallas import tpu as pltpu
from jax.experimental.pallas import tpu_sc as plsc

_EMBED = 64
_HID = 32
_BLK = 32768
_BLKQ = _BLK // 8


def _project_body(tT_ref, W1, b1, out_ref):
    w = W1[...].astype(jnp.bfloat16)
    parts = []
    for c in range(8):
        t = tT_ref[:, c * _BLKQ:(c + 1) * _BLKQ].astype(jnp.bfloat16)
        parts.append(
            lax.dot_general(t, w, (((0,), (0,)), ((), ())),
                            preferred_element_type=jnp.float32) + b1[...]
        )
    lo = jnp.concatenate(parts[0::2], axis=1)
    hi = jnp.concatenate(parts[1::2], axis=1)
    lo32 = lax.bitcast_convert_type(
        lo.astype(jnp.bfloat16).astype(jnp.float32), jnp.uint32)
    hi32 = lax.bitcast_convert_type(
        hi.astype(jnp.bfloat16).astype(jnp.float32), jnp.uint32)
    packed = (lo32 >> 16) | (hi32 & jnp.uint32(0xFFFF0000))
    out_ref[...] = lax.bitcast_convert_type(packed, jnp.int32)


@functools.lru_cache(maxsize=None)
def _make_project(V: int):
    grid = (V + _BLK - 1) // _BLK
    full = lambda shape: pl.BlockSpec(shape, lambda b: (0,) * len(shape))
    return pl.pallas_call(
        _project_body,
        grid=(grid,),
        in_specs=[
            pl.BlockSpec((_EMBED, _BLK), lambda b: (0, b)),
            full((_EMBED, _HID)),
            full((1, _HID)),
        ],
        out_specs=pl.BlockSpec((_BLKQ, 4 * _HID), lambda b: (b, 0)),
        out_shape=jax.ShapeDtypeStruct((grid * _BLKQ, 4 * _HID), jnp.int32),
    )


@functools.lru_cache(maxsize=None)
def _make_gather(B: int, D: int, V: int):
    info = plsc.get_sparse_core_info()
    nw = info.num_cores * info.num_subcores
    assert B % (8 * nw) == 0
    bpw = B // nw
    mesh = plsc.VectorSubcoreMesh(core_axis_name="c", subcore_axis_name="s")

    @functools.partial(
        pl.kernel,
        mesh=mesh,
        compiler_params=pltpu.CompilerParams(needs_layout_passes=False),
        out_type=jax.ShapeDtypeStruct((B, D), jnp.int32),
        scratch_types=[
            pltpu.VMEM((bpw,), jnp.int32),
            pltpu.VMEM((bpw, D), jnp.int32),
            pltpu.SemaphoreType.DMA,
        ],
    )
    def gather1(tbl, ids_hbm, out, idx_v, rows, sem):
        wid = lax.axis_index("s") * info.num_cores + lax.axis_index("c")
        base = wid * bpw
        lanes = lax.iota(jnp.int32, 16)
        pltpu.sync_copy(ids_hbm.at[pl.ds(base, bpw)], idx_v)

        def g_body(g, c):
            v = idx_v[pl.ds(g * 16, 16)]
            for l in range(16):
                s = jnp.sum(jnp.where(lanes == l, v, 0))
                pltpu.async_copy(tbl.at[s], rows.at[g * 16 + l], sem)
            return c

        lax.fori_loop(0, bpw // 16, g_body, 0)

        def d_body(j, c):
            pltpu.make_async_copy(tbl.at[0], rows.at[j], sem).wait()
            return c

        lax.fori_loop(0, bpw, d_body, 0)
        pltpu.sync_copy(rows, out.at[pl.ds(base, bpw)])

    return gather1


def _finish_body(u4_ref, i4_ref, us_ref, is_ref, uW2, ub2, iW2, ib2, out_ref):
    def select(p4, slot):
        x = lax.bitcast_convert_type(p4, jnp.uint32)
        lo = lax.bitcast_convert_type(x << 16, jnp.float32)
        hi = lax.bitcast_convert_type(x & jnp.uint32(0xFFFF0000), jnp.float32)
        acc = jnp.zeros(lo[:, :_HID].shape, jnp.float32)
        for q in range(8):
            half = lo if q % 2 == 0 else hi
            g = q // 2
            acc = acc + jnp.where(slot == q, half[:, g * _HID:(g + 1) * _HID], 0.0)
        return acc

    uh = jnp.maximum(select(u4_ref[...], us_ref[...]), 0.0)
    uv = jnp.dot(uh, uW2[...], preferred_element_type=jnp.float32) + ub2[...]
    ih = jnp.maximum(select(i4_ref[...], is_ref[...]), 0.0)
    iv = jnp.dot(ih, iW2[...], preferred_element_type=jnp.float32) + ib2[...]
    un = jnp.maximum(jnp.sqrt(jnp.sum(uv * uv, axis=1)), 1e-12)
    inn = jnp.maximum(jnp.sqrt(jnp.sum(iv * iv, axis=1)), 1e-12)
    out_ref[...] = jnp.sum(uv * iv, axis=1) / (un * inn)


@functools.lru_cache(maxsize=None)
def _make_finish(B: int, blk: int):
    grid = B // blk
    full = lambda shape: pl.BlockSpec(shape, lambda b: (0,) * len(shape))
    return pl.pallas_call(
        _finish_body,
        grid=(grid,),
        in_specs=[
            pl.BlockSpec((blk, 4 * _HID), lambda b: (b, 0)),
            pl.BlockSpec((blk, 4 * _HID), lambda b: (b, 0)),
            pl.BlockSpec((blk, 1), lambda b: (b, 0)),
            pl.BlockSpec((blk, 1), lambda b: (b, 0)),
            full((_HID, _HID)),
            full((1, _HID)),
            full((_HID, _HID)),
            full((1, _HID)),
        ],
        out_specs=pl.BlockSpec((blk,), lambda b: (b,)),
        out_shape=jax.ShapeDtypeStruct((B,), jnp.float32),
    )


def kernel(user_ids, item_ids, user_table, item_table,
           uW1, ub1, uW2, ub2, iW1, ib1, iW2, ib2):
    B = user_ids.shape[0]
    V = user_table.shape[0]
    uid = user_ids.astype(jnp.int32)
    iid = item_ids.astype(jnp.int32)
    urow = (uid // _BLK) * _BLKQ + (uid % _BLKQ)
    irow = (iid // _BLK) * _BLKQ + (iid % _BLKQ)
    uslot = (uid % _BLK) // _BLKQ
    islot = (iid % _BLK) // _BLKQ
    project = _make_project(V)
    gather = _make_gather(B, 4 * _HID, V)
    P_u = project(user_table.T, uW1, ub1.reshape(1, _HID))
    u4 = gather(P_u, urow)
    P_i = project(item_table.T, iW1, ib1.reshape(1, _HID))
    i4 = gather(P_i, irow)
    return _make_finish(B, 2048)(
        u4, i4, uslot.reshape(B, 1), islot.reshape(B, 1),
        uW2, ub2.reshape(1, _HID), iW2, ib2.reshape(1, _HID),
    )
